# baseline (speedup 1.0000x reference)
import functools

import jax
import jax.numpy as jnp
from jax import lax
from jax.experimental import pallas as pl
from jax.experimental.pallas import tpu as pltpu
from jax.experimental.pallas import tpu_sc as plsc

VOCAB = 10
DIM = 64
NUM_CORES = 2
NUM_SUBCORES = 16
NUM_WORKERS = NUM_CORES * NUM_SUBCORES

CHUNK = 512


def _make_sc_lookup(n_rows: int):
  per_w = n_rows // NUM_WORKERS
  n_chunks = per_w // CHUNK
  assert per_w % CHUNK == 0 and n_chunks % 2 == 0

  mesh = plsc.VectorSubcoreMesh(
      core_axis_name="c", subcore_axis_name="s",
      num_cores=NUM_CORES, num_subcores=NUM_SUBCORES)

  @functools.partial(
      pl.kernel,
      out_type=jax.ShapeDtypeStruct((n_rows, DIM), jnp.float32),
      mesh=mesh,
      compiler_params=pltpu.CompilerParams(
          needs_layout_passes=False, use_tc_tiling_on_sc=False),
      scratch_types=[
          pltpu.VMEM((CHUNK,), jnp.int32),
          pltpu.VMEM((CHUNK,), jnp.int32),
          pltpu.VMEM((CHUNK, DIM), jnp.float32),
          pltpu.VMEM((CHUNK, DIM), jnp.float32),
          pltpu.SemaphoreType.DMA,
          pltpu.SemaphoreType.DMA,
          pltpu.SemaphoreType.DMA,
          pltpu.SemaphoreType.DMA,
          pltpu.SemaphoreType.DMA,
          pltpu.SemaphoreType.DMA,
      ],
  )
  def lookup(idx_hbm, table_hbm, out_hbm, idx_v0, idx_v1, rows_v0, rows_v1,
             sem_i0, sem_i1, sem_g0, sem_g1, sem_o0, sem_o1):
    idx_v = (idx_v0, idx_v1)
    rows_v = (rows_v0, rows_v1)
    sem_i = (sem_i0, sem_i1)
    sem_g = (sem_g0, sem_g1)
    sem_o = (sem_o0, sem_o1)
    wid = lax.axis_index("s") * NUM_CORES + lax.axis_index("c")
    wbase = wid * per_w

    def start_in(g, b):
      off = pl.multiple_of(wbase + g * CHUNK, CHUNK)
      pltpu.async_copy(idx_hbm.at[pl.ds(off, CHUNK)], idx_v[b], sem_i[b])

    def wait_in(b):
      pltpu.make_async_copy(
          idx_hbm.at[pl.ds(0, CHUNK)], idx_v[b], sem_i[b]).wait()

    def start_gather(b):
      pltpu.async_copy(table_hbm.at[idx_v[b]], rows_v[b], sem_g[b])

    def wait_gather(b):
      pltpu.make_async_copy(
          table_hbm.at[idx_v[b]], rows_v[b], sem_g[b]).wait()

    def start_out(g, b):
      off = pl.multiple_of(wbase + g * CHUNK, CHUNK)
      pltpu.async_copy(rows_v[b], out_hbm.at[pl.ds(off, CHUNK)], sem_o[b])

    def wait_out(b):
      pltpu.make_async_copy(
          rows_v[b], out_hbm.at[pl.ds(0, CHUNK)], sem_o[b]).wait()

    start_in(0, 0)
    start_in(1, 1)

    def step(t, carry):
      for b in range(2):
        g = 2 * t + b
        wait_in(b)

        @pl.when(t > 0)
        def _():
          wait_out(b)

        start_gather(b)
        wait_gather(b)
        start_out(g, b)

        @pl.when(g + 2 < n_chunks)
        def _():
          start_in(g + 2, b)
      return carry

    lax.fori_loop(0, n_chunks // 2, step, 0)
    wait_out(0)
    wait_out(1)

  return lookup


def kernel(indices, table):
  b, s = indices.shape
  n = b * s
  idx_flat = indices.reshape(n).astype(jnp.int32)
  out = _make_sc_lookup(n)(idx_flat, table)
  return out.reshape(b, s, DIM)


# speedup vs baseline: 2.1085x; 2.1085x over previous
import functools

import jax
import jax.numpy as jnp
from jax import lax
from jax.experimental import pallas as pl
from jax.experimental.pallas import tpu as pltpu
from jax.experimental.pallas import tpu_sc as plsc

VOCAB = 10
DIM = 64
LANES = 16
NUM_CORES = 2
NUM_SUBCORES = 16
NUM_WORKERS = NUM_CORES * NUM_SUBCORES

CHUNK = 512
GROUPS = CHUNK // LANES


def _make_sc_lookup(n_rows: int):
  per_w = n_rows // NUM_WORKERS
  n_chunks = per_w // CHUNK
  assert per_w % CHUNK == 0 and n_chunks % 2 == 0

  mesh = plsc.VectorSubcoreMesh(
      core_axis_name="c", subcore_axis_name="s",
      num_cores=NUM_CORES, num_subcores=NUM_SUBCORES)

  @functools.partial(
      pl.kernel,
      out_type=jax.ShapeDtypeStruct((n_rows * DIM,), jnp.float32),
      mesh=mesh,
      compiler_params=pltpu.CompilerParams(needs_layout_passes=False),
      scratch_types=[
          pltpu.VMEM((VOCAB * DIM,), jnp.float32),
          pltpu.VMEM((GROUPS, LANES), jnp.int32),
          pltpu.VMEM((GROUPS, LANES), jnp.int32),
          pltpu.VMEM((CHUNK * DIM,), jnp.float32),
          pltpu.VMEM((CHUNK * DIM,), jnp.float32),
          pltpu.SemaphoreType.DMA,
          pltpu.SemaphoreType.DMA,
          pltpu.SemaphoreType.DMA,
          pltpu.SemaphoreType.DMA,
      ],
  )
  def lookup(idx_hbm, table_hbm, out_hbm, table_v, idx_v0, idx_v1,
             rows_v0, rows_v1, sem_i0, sem_i1, sem_o0, sem_o1):
    idx_v = (idx_v0, idx_v1)
    rows_v = (rows_v0, rows_v1)
    sem_i = (sem_i0, sem_i1)
    sem_o = (sem_o0, sem_o1)
    wid = lax.axis_index("s") * NUM_CORES + lax.axis_index("c")
    wbase = wid * per_w
    gbase = wbase // LANES

    pltpu.sync_copy(table_hbm, table_v)
    lane = lax.iota(jnp.int32, 16)

    def start_in(g, b):
      off = pl.multiple_of(gbase + g * GROUPS, GROUPS)
      pltpu.async_copy(
          idx_hbm.at[pl.ds(off, GROUPS)], idx_v[b], sem_i[b])

    def wait_in(b):
      pltpu.make_async_copy(
          idx_hbm.at[pl.ds(0, GROUPS)], idx_v[b], sem_i[b]).wait()

    def start_out(g, b):
      off = pl.multiple_of((wbase + g * CHUNK) * DIM, CHUNK * DIM)
      pltpu.async_copy(
          rows_v[b], out_hbm.at[pl.ds(off, CHUNK * DIM)], sem_o[b])

    def wait_out(b):
      pltpu.make_async_copy(
          rows_v[b], out_hbm.at[pl.ds(0, CHUNK * DIM)], sem_o[b]).wait()

    def compute(b):
      rv = rows_v[b]
      iv = idx_v[b]

      @plsc.parallel_loop(0, GROUPS, unroll=2)
      def group(gi):
        ivec = iv[gi]
        src_base = ivec * DIM
        dst_base = (lane + gi * LANES) * DIM
        for kb in range(0, DIM, LANES):
          vals = [plsc.load_gather(table_v, [src_base + (kb + j)])
                  for j in range(LANES)]
          for j in range(LANES):
            plsc.store_scatter(rv, [dst_base + (kb + j)], vals[j])

    start_in(0, 0)
    start_in(1, 1)

    def step(t, carry):
      for b in range(2):
        g = 2 * t + b
        wait_in(b)

        @pl.when(t > 0)
        def _():
          wait_out(b)

        compute(b)
        start_out(g, b)

        @pl.when(g + 2 < n_chunks)
        def _():
          start_in(g + 2, b)
      return carry

    lax.fori_loop(0, n_chunks // 2, step, 0)
    wait_out(0)
    wait_out(1)

  return lookup


def kernel(indices, table):
  b, s = indices.shape
  n = b * s
  idx2d = indices.reshape(n // LANES, LANES).astype(jnp.int32)
  out = _make_sc_lookup(n)(idx2d, table.reshape(VOCAB * DIM))
  return out.reshape(b, s, DIM)


# speedup vs baseline: 6.8086x; 3.2292x over previous
import functools

import jax
import jax.numpy as jnp
from jax import lax
from jax.experimental import pallas as pl
from jax.experimental.pallas import tpu as pltpu
from jax.experimental.pallas import tpu_sc as plsc

VOCAB = 10
DIM = 64
LANES = 16
NUM_CORES = 2
NUM_SUBCORES = 16
NUM_WORKERS = NUM_CORES * NUM_SUBCORES

CHUNK = 512
GROUPS = CHUNK // LANES


def _make_sc_lookup(n_rows: int):
  per_w = n_rows // NUM_WORKERS
  n_chunks = per_w // CHUNK
  assert per_w % CHUNK == 0 and n_chunks % 2 == 0

  mesh = plsc.VectorSubcoreMesh(
      core_axis_name="c", subcore_axis_name="s",
      num_cores=NUM_CORES, num_subcores=NUM_SUBCORES)

  @functools.partial(
      pl.kernel,
      out_type=jax.ShapeDtypeStruct((n_rows, DIM), jnp.float32),
      mesh=mesh,
      compiler_params=pltpu.CompilerParams(
          needs_layout_passes=False, use_tc_tiling_on_sc=False),
      scratch_types=[
          pltpu.VMEM((VOCAB * DIM,), jnp.float32),
          pltpu.VMEM((GROUPS, LANES), jnp.int32),
          pltpu.VMEM((GROUPS, LANES), jnp.int32),
          pltpu.VMEM((CHUNK, DIM), jnp.float32),
          pltpu.VMEM((CHUNK, DIM), jnp.float32),
          pltpu.SemaphoreType.DMA,
          pltpu.SemaphoreType.DMA,
          pltpu.SemaphoreType.DMA,
          pltpu.SemaphoreType.DMA,
      ],
  )
  def lookup(idx_hbm, table_hbm, out_hbm, table_v, idx_v0, idx_v1,
             rows_v0, rows_v1, sem_i0, sem_i1, sem_o0, sem_o1):
    idx_v = (idx_v0, idx_v1)
    rows_v = (rows_v0, rows_v1)
    sem_i = (sem_i0, sem_i1)
    sem_o = (sem_o0, sem_o1)
    wid = lax.axis_index("s") * NUM_CORES + lax.axis_index("c")
    wbase = wid * per_w
    gbase = wbase // LANES

    pltpu.sync_copy(table_hbm, table_v)
    lane = lax.iota(jnp.int32, 16)

    def start_in(g, b):
      off = pl.multiple_of(gbase + g * GROUPS, GROUPS)
      pltpu.async_copy(
          idx_hbm.at[pl.ds(off, GROUPS)], idx_v[b], sem_i[b])

    def wait_in(b):
      pltpu.make_async_copy(
          idx_hbm.at[pl.ds(0, GROUPS)], idx_v[b], sem_i[b]).wait()

    def start_out(g, b):
      off = pl.multiple_of(wbase + g * CHUNK, CHUNK)
      pltpu.async_copy(rows_v[b], out_hbm.at[pl.ds(off, CHUNK)], sem_o[b])

    def wait_out(b):
      pltpu.make_async_copy(
          rows_v[b], out_hbm.at[pl.ds(0, CHUNK)], sem_o[b]).wait()

    def compute(b):
      rv = rows_v[b]
      iv = idx_v[b]

      @plsc.parallel_loop(0, GROUPS)
      def group(gi):
        ivec64 = iv[gi] * DIM
        for i in range(LANES):
          base = lax.gather(
              ivec64,
              jnp.full((LANES, 1), i, jnp.int32),
              lax.GatherDimensionNumbers(
                  offset_dims=(), collapsed_slice_dims=(0,),
                  start_index_map=(0,)),
              (1,),
              mode=lax.GatherScatterMode.PROMISE_IN_BOUNDS)
          dst = gi * LANES + i
          for j in range(DIM // LANES):
            vals = plsc.load_gather(table_v, [base + (j * LANES) + lane])
            rv[dst, pl.ds(j * LANES, LANES)] = vals

      del group

    start_in(0, 0)
    start_in(1, 1)

    def step(t, carry):
      for b in range(2):
        g = 2 * t + b
        wait_in(b)

        @pl.when(t > 0)
        def _():
          wait_out(b)

        compute(b)
        start_out(g, b)

        @pl.when(g + 2 < n_chunks)
        def _():
          start_in(g + 2, b)
      return carry

    lax.fori_loop(0, n_chunks // 2, step, 0)
    wait_out(0)
    wait_out(1)

  return lookup


def kernel(indices, table):
  b, s = indices.shape
  n = b * s
  idx2d = indices.reshape(n // LANES, LANES).astype(jnp.int32)
  out = _make_sc_lookup(n)(idx2d, table.reshape(VOCAB * DIM))
  return out.reshape(b, s, DIM)


# speedup vs baseline: 6.9411x; 1.0195x over previous
import functools

import jax
import jax.numpy as jnp
from jax import lax
from jax.experimental import pallas as pl
from jax.experimental.pallas import tpu as pltpu
from jax.experimental.pallas import tpu_sc as plsc

VOCAB = 10
DIM = 64
LANES = 16
NUM_CORES = 2
NUM_SUBCORES = 16
NUM_WORKERS = NUM_CORES * NUM_SUBCORES

RCHUNK = 4


def _make_sc_lookup(n_batch: int, seq: int):
  per_w = n_batch // NUM_WORKERS
  n_chunks = per_w // RCHUNK
  nlook = RCHUNK * seq
  n_groups = nlook // LANES
  assert per_w % RCHUNK == 0 and n_chunks % 2 == 0 and nlook % LANES == 0

  mesh = plsc.VectorSubcoreMesh(
      core_axis_name="c", subcore_axis_name="s",
      num_cores=NUM_CORES, num_subcores=NUM_SUBCORES)

  @functools.partial(
      pl.kernel,
      out_type=jax.ShapeDtypeStruct((n_batch, seq, DIM), jnp.float32),
      mesh=mesh,
      compiler_params=pltpu.CompilerParams(
          needs_layout_passes=False, use_tc_tiling_on_sc=False),
      scratch_types=[
          pltpu.VMEM((VOCAB * DIM,), jnp.float32),
          pltpu.VMEM((RCHUNK, seq), jnp.int32),
          pltpu.VMEM((RCHUNK, seq), jnp.int32),
          pltpu.VMEM((RCHUNK, seq, DIM), jnp.float32),
          pltpu.VMEM((RCHUNK, seq, DIM), jnp.float32),
          pltpu.SemaphoreType.DMA,
          pltpu.SemaphoreType.DMA,
          pltpu.SemaphoreType.DMA,
          pltpu.SemaphoreType.DMA,
      ],
  )
  def lookup(idx_hbm, table_hbm, out_hbm, table_v, idx_v0, idx_v1,
             rows_v0, rows_v1, sem_i0, sem_i1, sem_o0, sem_o1):
    idx_v = (idx_v0, idx_v1)
    rows_v = (rows_v0, rows_v1)
    sem_i = (sem_i0, sem_i1)
    sem_o = (sem_o0, sem_o1)
    wid = lax.axis_index("s") * NUM_CORES + lax.axis_index("c")
    wrow = wid * per_w

    pltpu.sync_copy(table_hbm, table_v)
    lane = lax.iota(jnp.int32, 16)

    def start_in(g, b):
      off = pl.multiple_of(wrow + g * RCHUNK, RCHUNK)
      pltpu.async_copy(idx_hbm.at[pl.ds(off, RCHUNK)], idx_v[b], sem_i[b])

    def wait_in(b):
      pltpu.make_async_copy(
          idx_hbm.at[pl.ds(0, RCHUNK)], idx_v[b], sem_i[b]).wait()

    def start_out(g, b):
      off = pl.multiple_of(wrow + g * RCHUNK, RCHUNK)
      pltpu.async_copy(rows_v[b], out_hbm.at[pl.ds(off, RCHUNK)], sem_o[b])

    def wait_out(b):
      pltpu.make_async_copy(
          rows_v[b], out_hbm.at[pl.ds(0, RCHUNK)], sem_o[b]).wait()

    def bcast(vec, i):
      return lax.gather(
          vec,
          jnp.full((LANES, 1), i, jnp.int32),
          lax.GatherDimensionNumbers(
              offset_dims=(), collapsed_slice_dims=(0,),
              start_index_map=(0,)),
          (1,),
          mode=lax.GatherScatterMode.PROMISE_IN_BOUNDS)

    def compute(b):
      rv = rows_v[b]
      iv = idx_v[b]

      @plsc.parallel_loop(0, n_groups)
      def group(gi):
        pvec = gi * LANES + lane
        ivec64 = plsc.load_gather(
            iv, [pvec // seq, pvec % seq]) * DIM
        for i in range(LANES):
          base = bcast(ivec64, i)
          p = gi * LANES + i
          r = p // seq
          c = p % seq
          for j in range(DIM // LANES):
            vals = plsc.load_gather(table_v, [base + (j * LANES) + lane])
            rv[r, c, pl.ds(j * LANES, LANES)] = vals

      del group

    start_in(0, 0)
    start_in(1, 1)

    def step(t, carry):
      for b in range(2):
        g = 2 * t + b
        wait_in(b)

        @pl.when(t > 0)
        def _():
          wait_out(b)

        compute(b)
        start_out(g, b)

        @pl.when(g + 2 < n_chunks)
        def _():
          start_in(g + 2, b)
      return carry

    lax.fori_loop(0, n_chunks // 2, step, 0)
    wait_out(0)
    wait_out(1)

  return lookup


def kernel(indices, table):
  b, s = indices.shape
  return _make_sc_lookup(b, s)(
      indices.astype(jnp.int32), table.reshape(VOCAB * DIM))


# speedup vs baseline: 9.0710x; 1.3069x over previous
import functools

import jax
import jax.numpy as jnp
from jax import lax
from jax.experimental import pallas as pl
from jax.experimental.pallas import tpu as pltpu
from jax.experimental.pallas import tpu_sc as plsc

VOCAB = 10
DIM = 64
LANES = 16
NUM_CORES = 2
NUM_SUBCORES = 16
NUM_WORKERS = NUM_CORES * NUM_SUBCORES

SUPER = 8
RCHUNK = 2
SUBS = SUPER // RCHUNK


def _make_sc_lookup(n_batch: int, seq: int):
  per_w = n_batch // NUM_WORKERS
  n_blocks = per_w // SUPER
  nlook = RCHUNK * seq
  n_groups = nlook // LANES
  assert per_w % SUPER == 0 and n_blocks % 2 == 0 and nlook % LANES == 0

  mesh = plsc.VectorSubcoreMesh(
      core_axis_name="c", subcore_axis_name="s",
      num_cores=NUM_CORES, num_subcores=NUM_SUBCORES)

  @functools.partial(
      pl.kernel,
      out_type=jax.ShapeDtypeStruct((n_batch, seq, DIM), jnp.float32),
      mesh=mesh,
      compiler_params=pltpu.CompilerParams(needs_layout_passes=False),
      scratch_types=[
          pltpu.VMEM((VOCAB * DIM,), jnp.float32),
          pltpu.VMEM((SUPER, seq), jnp.int32),
          pltpu.VMEM((SUPER, seq), jnp.int32),
          pltpu.VMEM((RCHUNK, seq, DIM), jnp.float32),
          pltpu.VMEM((RCHUNK, seq, DIM), jnp.float32),
          pltpu.SemaphoreType.DMA,
          pltpu.SemaphoreType.DMA,
          pltpu.SemaphoreType.DMA,
          pltpu.SemaphoreType.DMA,
      ],
  )
  def lookup(idx_hbm, table_hbm, out_hbm, table_v, idx_v0, idx_v1,
             rows_v0, rows_v1, sem_i0, sem_i1, sem_o0, sem_o1):
    idx_v = (idx_v0, idx_v1)
    rows_v = (rows_v0, rows_v1)
    sem_i = (sem_i0, sem_i1)
    sem_o = (sem_o0, sem_o1)
    wid = lax.axis_index("s") * NUM_CORES + lax.axis_index("c")
    wrow = wid * per_w

    pltpu.sync_copy(table_hbm, table_v)
    lane = lax.iota(jnp.int32, 16)

    def start_in(blk, b):
      off = pl.multiple_of(wrow + blk * SUPER, SUPER)
      pltpu.async_copy(idx_hbm.at[pl.ds(off, SUPER)], idx_v[b], sem_i[b])

    def wait_in(b):
      pltpu.make_async_copy(
          idx_hbm.at[pl.ds(0, SUPER)], idx_v[b], sem_i[b]).wait()

    def start_out(blk, sub, b):
      off = pl.multiple_of(wrow + blk * SUPER + sub * RCHUNK, RCHUNK)
      pltpu.async_copy(rows_v[b], out_hbm.at[pl.ds(off, RCHUNK)], sem_o[b])

    def wait_out(b):
      pltpu.make_async_copy(
          rows_v[b], out_hbm.at[pl.ds(0, RCHUNK)], sem_o[b]).wait()

    def bcast(vec, i):
      return lax.gather(
          vec,
          jnp.full((LANES, 1), i, jnp.int32),
          lax.GatherDimensionNumbers(
              offset_dims=(), collapsed_slice_dims=(0,),
              start_index_map=(0,)),
          (1,),
          mode=lax.GatherScatterMode.PROMISE_IN_BOUNDS)

    def compute(pb, sub, b):
      rv = rows_v[b]
      iv = idx_v[pb]

      @plsc.parallel_loop(0, n_groups)
      def group(gi):
        pvec = gi * LANES + lane
        ivec64 = plsc.load_gather(
            iv, [pvec // seq + (sub * RCHUNK), pvec % seq]) * DIM
        for i in range(LANES):
          base = bcast(ivec64, i)
          p = gi * LANES + i
          r = p // seq
          c = p % seq
          for j in range(DIM // LANES):
            vals = plsc.load_gather(table_v, [base + (j * LANES) + lane])
            rv[r, c, pl.ds(j * LANES, LANES)] = vals

      del group

    start_in(0, 0)
    start_in(1, 1)

    def step(t, carry):
      for pb in range(2):
        blk = 2 * t + pb
        wait_in(pb)
        for sub in range(SUBS):
          b = sub % 2
          if sub >= 2:
            wait_out(b)
          else:
            @pl.when(blk > 0)
            def _():
              wait_out(b)

          compute(pb, sub, b)
          start_out(blk, sub, b)

        @pl.when(blk + 2 < n_blocks)
        def _():
          start_in(blk + 2, pb)
      return carry

    lax.fori_loop(0, n_blocks // 2, step, 0)
    wait_out(0)
    wait_out(1)

  return lookup


def kernel(indices, table):
  b, s = indices.shape
  return _make_sc_lookup(b, s)(
      indices.astype(jnp.int32), table.reshape(VOCAB * DIM))


# speedup vs baseline: 42.4794x; 4.6830x over previous
import functools

import jax
import jax.numpy as jnp
from jax import lax
from jax.experimental import pallas as pl
from jax.experimental.pallas import tpu as pltpu
from jax.experimental.pallas import tpu_sc as plsc

VOCAB = 10
DIM = 64
LANES = 16
REP = LANES
NUM_CORES = 2
NUM_SUBCORES = 16
NUM_WORKERS = NUM_CORES * NUM_SUBCORES

SBLK = 8
BCHUNK = 512


def _make_sc_lookup(n_batch: int, seq: int):
  n_sblk = seq // SBLK
  n_bc = n_batch // BCHUNK
  n_tasks = n_sblk * n_bc
  per_w = n_tasks // NUM_WORKERS
  n_bg = BCHUNK // LANES
  assert n_tasks % NUM_WORKERS == 0 and seq % SBLK == 0

  mesh = plsc.VectorSubcoreMesh(
      core_axis_name="c", subcore_axis_name="s",
      num_cores=NUM_CORES, num_subcores=NUM_SUBCORES)

  @functools.partial(
      pl.kernel,
      out_type=jax.ShapeDtypeStruct((seq, DIM, n_batch), jnp.float32),
      mesh=mesh,
      compiler_params=pltpu.CompilerParams(needs_layout_passes=False),
      scratch_types=[
          pltpu.VMEM((VOCAB * DIM,), jnp.float32),
          pltpu.VMEM((DIM * VOCAB * REP,), jnp.float32),
          pltpu.VMEM((SBLK, BCHUNK), jnp.int32),
          pltpu.VMEM((DIM, BCHUNK), jnp.float32),
          pltpu.VMEM((DIM, BCHUNK), jnp.float32),
          pltpu.SemaphoreType.DMA,
          pltpu.SemaphoreType.DMA,
      ],
  )
  def lookup(idx_hbm, table_hbm, out_hbm, table_v, rep_v, idx_v,
             plane_v0, plane_v1, sem_o0, sem_o1):
    plane_v = (plane_v0, plane_v1)
    sem_o = (sem_o0, sem_o1)
    wid = lax.axis_index("s") * NUM_CORES + lax.axis_index("c")

    pltpu.sync_copy(table_hbm, table_v)
    lane = lax.iota(jnp.int32, 16)

    def build(d, carry):
      for v in range(VOCAB):
        tvec = plsc.load_gather(
            table_v, [jnp.full((LANES,), v * DIM, jnp.int32) + d])
        rep_v[pl.ds(d * (VOCAB * REP) + v * REP, LANES)] = tvec
      return carry

    lax.fori_loop(0, DIM, build, 0)

    def start_out(s_glob, b0, b):
      pltpu.async_copy(
          plane_v[b], out_hbm.at[s_glob, :, pl.ds(b0, BCHUNK)], sem_o[b])

    def wait_out(b):
      pltpu.make_async_copy(
          plane_v[b], out_hbm.at[0, :, pl.ds(0, BCHUNK)], sem_o[b]).wait()

    zeros = jnp.zeros((LANES,), jnp.int32)

    def task(sblk, carry):
      k = sblk
      b0 = pl.multiple_of(wid * BCHUNK, BCHUNK)
      pltpu.sync_copy(
          idx_hbm.at[sblk, :, pl.ds(b0, BCHUNK)], idx_v)

      for sl in range(SBLK):
        b = sl % 2
        if sl >= 2:
          wait_out(b)
        else:
          @pl.when(k > 0)
          def _():
            wait_out(b)

        rv = plane_v[b]
        svec = jnp.full((LANES,), sl, jnp.int32)

        addrs = []
        for bg in range(n_bg):
          ivec = plsc.load_gather(idx_v, [svec, bg * LANES + lane])
          addrs.append(ivec * REP + lane)

        @plsc.parallel_loop(0, DIM)
        def drow(d):
          doff = d * (VOCAB * REP)
          for bg in range(n_bg):
            vals = plsc.load_gather(rep_v, [addrs[bg] + doff])
            rv[d, pl.ds(bg * LANES, LANES)] = vals

        del drow
        start_out(sblk * SBLK + sl, b0, b)
      return carry

    lax.fori_loop(0, n_sblk, task, 0)
    wait_out(0)
    wait_out(1)

  return lookup


def kernel(indices, table):
  b, s = indices.shape
  idx_t = jnp.transpose(indices.astype(jnp.int32), (1, 0))
  idx_blocks = idx_t.reshape(s // SBLK, SBLK, b)
  out_t = _make_sc_lookup(b, s)(idx_blocks, table.reshape(VOCAB * DIM))
  return jnp.transpose(out_t, (2, 0, 1))
